# baseline (speedup 1.0000x reference)
import functools

import jax
import jax.numpy as jnp
from jax import lax
from jax.experimental import pallas as pl
from jax.experimental.pallas import tpu as pltpu
from jax.experimental.pallas import tpu_sc as plsc

G_ = 0.99
N_ = 2 * 4096 * 2048
NW_ = 32
SHARD_ = N_ // NW_
CHUNK_ = 16384
NCHUNK_ = SHARD_ // CHUNK_
HB_ = 65536

K99_99_ = int(round(0.9999 * N_)) - 1
K99_9_ = int(round(0.999 * N_)) - 1
K99_ = int(round(0.99 * N_)) - 1

_MESH = plsc.VectorSubcoreMesh(
    core_axis_name="c", subcore_axis_name="s", num_cores=2, num_subcores=16
)


def _zero_vmem(ref, nwords):
    z = jnp.zeros((16,), jnp.int32)

    def body(j, _):
        ref[pl.ds(j * 16, 16)] = z
        return 0

    lax.fori_loop(0, nwords // 16, body, 0, unroll=8)


def _start(x_hbm, buf_v, sem, base, ci):
    off = jnp.minimum(base + ci * CHUNK_, N_ - CHUNK_)
    pltpu.async_copy(x_hbm.at[pl.ds(off, CHUNK_)], buf_v, sem)


def _wait(x_hbm, buf_v, sem):
    pltpu.make_async_copy(x_hbm.at[pl.ds(0, CHUNK_)], buf_v, sem).wait()


@functools.partial(
    pl.kernel,
    out_type=(
        jax.ShapeDtypeStruct((NW_, HB_), jnp.int32),
        jax.ShapeDtypeStruct((NW_, 16), jnp.int32),
    ),
    mesh=_MESH,
    compiler_params=pltpu.CompilerParams(needs_layout_passes=False),
    scratch_types=[
        pltpu.VMEM((2, CHUNK_), jnp.float32),
        pltpu.VMEM((HB_,), jnp.int32),
        pltpu.VMEM((16,), jnp.int32),
        pltpu.SemaphoreType.DMA,
        pltpu.SemaphoreType.DMA,
    ],
)
def _sc_hist(x_hbm, hist_hbm, max_hbm, buf_v, hist_v, max_v, sem0, sem1):
    wid = lax.axis_index("s") * 2 + lax.axis_index("c")
    base = wid * SHARD_
    sems = (sem0, sem1)
    _start(x_hbm, buf_v.at[0], sem0, base, 0)
    _start(x_hbm, buf_v.at[1], sem1, base, 1)
    _zero_vmem(hist_v, HB_)
    ones = jnp.ones((16,), jnp.int32)

    def chunk_body(g, maxv):
        for b in range(2):
            ci = 2 * g + b
            _wait(x_hbm, buf_v.at[b], sems[b])

            def vbody(i, mv, b=b):
                v = buf_v[b, pl.ds(i * 16, 16)]
                u = lax.bitcast_convert_type(v, jnp.int32) & 0x7FFFFFFF
                plsc.addupdate_scatter(hist_v, [u >> 15], ones)
                return jnp.maximum(mv, u)

            maxv = lax.fori_loop(0, CHUNK_ // 16, vbody, maxv, unroll=8)
            _start(x_hbm, buf_v.at[b], sems[b], base, ci + 2)
        return maxv

    maxv = lax.fori_loop(0, NCHUNK_ // 2, chunk_body, jnp.zeros((16,), jnp.int32))
    _wait(x_hbm, buf_v.at[0], sem0)
    _wait(x_hbm, buf_v.at[1], sem1)
    max_v[...] = maxv
    pltpu.sync_copy(hist_v, hist_hbm.at[wid])
    pltpu.sync_copy(max_v, max_hbm.at[wid])


def _cum_lanes(a):
    c = a
    sh = 1
    while sh < a.shape[-1]:
        z = jnp.zeros(a.shape[:-1] + (sh,), a.dtype)
        c = c + jnp.concatenate([z, c[..., :-sh]], axis=-1)
        sh *= 2
    return c


def _cum_rows(a):
    c = a
    sh = 1
    while sh < a.shape[0]:
        z = jnp.zeros((sh,) + a.shape[1:], a.dtype)
        c = c + jnp.concatenate([z, c[:-sh]], axis=0)
        sh *= 2
    return c


def _glue_body(hist_ref, max_ref, bufs_ref, o_ref):
    h = jnp.sum(hist_ref[...], axis=0)
    a = h.reshape(512, 128)
    rowcum = _cum_lanes(a)
    rs = rowcum[:, 127:]
    cum = rowcum + _cum_rows(rs) - rs
    maxu = jnp.max(max_ref[...])

    bits = [maxu]
    for kk in (K99_99_, K99_9_, K99_):
        b = jnp.sum((cum <= kk).astype(jnp.int32))
        bits.append((b << 15) | 0x4000)

    c = lax.broadcasted_iota(jnp.int32, (1, 128), 1)
    vbits = jnp.zeros((1, 128), jnp.int32)
    for t, w in enumerate(bits):
        vbits = jnp.where(c == t, w, vbits)
    vals = lax.bitcast_convert_type(vbits, jnp.float32)
    o_ref[...] = bufs_ref[...] * jnp.float32(G_) + vals * jnp.float32(1.0 - G_)


def kernel(x, max_buf, p99_99_buf, p99_9_buf, p99_buf):
    xf = x.reshape(-1)
    hist, maxes = _sc_hist(xf)
    bufs = jnp.zeros((1, 128), jnp.float32)
    bufs = bufs.at[0, 0].set(max_buf)
    bufs = bufs.at[0, 1].set(p99_99_buf)
    bufs = bufs.at[0, 2].set(p99_9_buf)
    bufs = bufs.at[0, 3].set(p99_buf)
    outv = pl.pallas_call(
        _glue_body,
        out_shape=jax.ShapeDtypeStruct((1, 128), jnp.float32),
    )(hist, maxes, bufs)
    return (x, outv[0, 0], outv[0, 1], outv[0, 2], outv[0, 3])


# speedup vs baseline: 2.0580x; 2.0580x over previous
import functools

import jax
import jax.numpy as jnp
from jax import lax
from jax.experimental import pallas as pl
from jax.experimental.pallas import tpu as pltpu
from jax.experimental.pallas import tpu_sc as plsc

G_ = 0.99
N_ = 2 * 4096 * 2048
NW_ = 32
SHARD_ = N_ // NW_
CHUNK_ = 16384
NCHUNK_ = SHARD_ // CHUNK_
HB_ = 65536

K99_99_ = int(round(0.9999 * N_)) - 1
K99_9_ = int(round(0.999 * N_)) - 1
K99_ = int(round(0.99 * N_)) - 1

_MESH = plsc.VectorSubcoreMesh(
    core_axis_name="c", subcore_axis_name="s", num_cores=2, num_subcores=16
)


def _zero_vmem(ref, nwords):
    z = jnp.zeros((16,), jnp.int32)

    def body(j, _):
        ref[pl.ds(j * 16, 16)] = z
        return 0

    lax.fori_loop(0, nwords // 16, body, 0, unroll=8)


def _start(x_hbm, buf_v, sem, base, ci):
    off = jnp.minimum(base + ci * CHUNK_, N_ - CHUNK_)
    pltpu.async_copy(x_hbm.at[pl.ds(off, CHUNK_)], buf_v, sem)


def _wait(x_hbm, buf_v, sem):
    pltpu.make_async_copy(x_hbm.at[pl.ds(0, CHUNK_)], buf_v, sem).wait()


@functools.partial(
    pl.kernel,
    out_type=(
        jax.ShapeDtypeStruct((NW_, HB_), jnp.int32),
        jax.ShapeDtypeStruct((NW_, 16), jnp.int32),
    ),
    mesh=_MESH,
    compiler_params=pltpu.CompilerParams(needs_layout_passes=False),
    scratch_types=[
        pltpu.VMEM((2, CHUNK_), jnp.float32),
        pltpu.VMEM((HB_,), jnp.int32),
        pltpu.VMEM((16,), jnp.int32),
        pltpu.SemaphoreType.DMA,
        pltpu.SemaphoreType.DMA,
    ],
)
def _sc_hist(x_hbm, hist_hbm, max_hbm, buf_v, hist_v, max_v, sem0, sem1):
    wid = lax.axis_index("s") * 2 + lax.axis_index("c")
    base = wid * SHARD_
    sems = (sem0, sem1)
    _start(x_hbm, buf_v.at[0], sem0, base, 0)
    _start(x_hbm, buf_v.at[1], sem1, base, 1)
    _zero_vmem(hist_v, HB_)
    ones = jnp.ones((16,), jnp.int32)

    def chunk_body(g, maxv):
        for b in range(2):
            ci = 2 * g + b
            _wait(x_hbm, buf_v.at[b], sems[b])

            def vbody(i, mv, b=b):
                v = buf_v[b, pl.ds(i * 16, 16)]
                u = lax.bitcast_convert_type(v, jnp.int32) & 0x7FFFFFFF
                plsc.addupdate_scatter(hist_v, [u >> 15], ones)
                return jnp.maximum(mv, u)

            maxv = plsc.parallel_loop(
                0, CHUNK_ // 16, 1, unroll=8, carry=maxv
            )(vbody)
            _start(x_hbm, buf_v.at[b], sems[b], base, ci + 2)
        return maxv

    maxv = lax.fori_loop(0, NCHUNK_ // 2, chunk_body, jnp.zeros((16,), jnp.int32))
    _wait(x_hbm, buf_v.at[0], sem0)
    _wait(x_hbm, buf_v.at[1], sem1)
    max_v[...] = maxv
    pltpu.sync_copy(hist_v, hist_hbm.at[wid])
    pltpu.sync_copy(max_v, max_hbm.at[wid])


def _cum_lanes(a):
    c = a
    sh = 1
    while sh < a.shape[-1]:
        z = jnp.zeros(a.shape[:-1] + (sh,), a.dtype)
        c = c + jnp.concatenate([z, c[..., :-sh]], axis=-1)
        sh *= 2
    return c


def _cum_rows(a):
    c = a
    sh = 1
    while sh < a.shape[0]:
        z = jnp.zeros((sh,) + a.shape[1:], a.dtype)
        c = c + jnp.concatenate([z, c[:-sh]], axis=0)
        sh *= 2
    return c


def _glue_body(hist_ref, max_ref, bufs_ref, o_ref):
    h = jnp.sum(hist_ref[...], axis=0)
    a = h.reshape(512, 128)
    rowcum = _cum_lanes(a)
    rs = rowcum[:, 127:]
    cum = rowcum + _cum_rows(rs) - rs
    maxu = jnp.max(max_ref[...])

    bits = [maxu]
    for kk in (K99_99_, K99_9_, K99_):
        b = jnp.sum((cum <= kk).astype(jnp.int32))
        bits.append((b << 15) | 0x4000)

    c = lax.broadcasted_iota(jnp.int32, (1, 128), 1)
    vbits = jnp.zeros((1, 128), jnp.int32)
    for t, w in enumerate(bits):
        vbits = jnp.where(c == t, w, vbits)
    vals = lax.bitcast_convert_type(vbits, jnp.float32)
    o_ref[...] = bufs_ref[...] * jnp.float32(G_) + vals * jnp.float32(1.0 - G_)


def kernel(x, max_buf, p99_99_buf, p99_9_buf, p99_buf):
    xf = x.reshape(-1)
    hist, maxes = _sc_hist(xf)
    bufs = jnp.zeros((1, 128), jnp.float32)
    bufs = bufs.at[0, 0].set(max_buf)
    bufs = bufs.at[0, 1].set(p99_99_buf)
    bufs = bufs.at[0, 2].set(p99_9_buf)
    bufs = bufs.at[0, 3].set(p99_buf)
    outv = pl.pallas_call(
        _glue_body,
        out_shape=jax.ShapeDtypeStruct((1, 128), jnp.float32),
    )(hist, maxes, bufs)
    return (x, outv[0, 0], outv[0, 1], outv[0, 2], outv[0, 3])


# speedup vs baseline: 2.5249x; 1.2268x over previous
import functools

import jax
import jax.numpy as jnp
from jax import lax
from jax.experimental import pallas as pl
from jax.experimental.pallas import tpu as pltpu
from jax.experimental.pallas import tpu_sc as plsc

G_ = 0.99
N_ = 2 * 4096 * 2048
NW_ = 32
SHARD_ = N_ // NW_
CHUNK_ = 16384
NCHUNK_ = SHARD_ // CHUNK_
HB_ = 65536

K99_99_ = int(round(0.9999 * N_)) - 1
K99_9_ = int(round(0.999 * N_)) - 1
K99_ = int(round(0.99 * N_)) - 1

_MESH = plsc.VectorSubcoreMesh(
    core_axis_name="c", subcore_axis_name="s", num_cores=2, num_subcores=16
)


def _zero_vmem(ref, nwords):
    z = jnp.zeros((16,), jnp.int32)

    def body(j, _):
        ref[pl.ds(j * 16, 16)] = z
        return 0

    lax.fori_loop(0, nwords // 16, body, 0, unroll=8)


def _start(x_hbm, buf_v, sem, base, ci):
    off = jnp.minimum(base + ci * CHUNK_, N_ - CHUNK_)
    pltpu.async_copy(x_hbm.at[pl.ds(off, CHUNK_)], buf_v, sem)


def _wait(x_hbm, buf_v, sem):
    pltpu.make_async_copy(x_hbm.at[pl.ds(0, CHUNK_)], buf_v, sem).wait()


@functools.partial(
    pl.kernel,
    out_type=(
        jax.ShapeDtypeStruct((NW_, HB_), jnp.int32),
        jax.ShapeDtypeStruct((NW_, 16), jnp.int32),
    ),
    mesh=_MESH,
    compiler_params=pltpu.CompilerParams(needs_layout_passes=False),
    scratch_types=[
        pltpu.VMEM((2, CHUNK_), jnp.float32),
        pltpu.VMEM((HB_,), jnp.int32),
        pltpu.VMEM((16,), jnp.int32),
        pltpu.SemaphoreType.DMA,
        pltpu.SemaphoreType.DMA,
    ],
)
def _sc_hist(x_hbm, hist_hbm, max_hbm, buf_v, hist_v, max_v, sem0, sem1):
    wid = lax.axis_index("s") * 2 + lax.axis_index("c")
    base = wid * SHARD_
    sems = (sem0, sem1)
    _start(x_hbm, buf_v.at[0], sem0, base, 0)
    _start(x_hbm, buf_v.at[1], sem1, base, 1)
    _zero_vmem(hist_v, HB_)
    ones = jnp.ones((16,), jnp.int32)

    def chunk_body(g, maxv):
        for b in range(2):
            ci = 2 * g + b
            _wait(x_hbm, buf_v.at[b], sems[b])

            def vbody(i, mv, b=b):
                v = buf_v[b, pl.ds(i * 16, 16)]
                u = lax.bitcast_convert_type(v, jnp.int32) & 0x7FFFFFFF
                plsc.addupdate_scatter(hist_v, [u >> 15], ones)
                return jnp.maximum(mv, u)

            maxv = plsc.parallel_loop(
                0, CHUNK_ // 16, 1, unroll=8, carry=maxv
            )(vbody)
            _start(x_hbm, buf_v.at[b], sems[b], base, ci + 2)
        return maxv

    maxv = lax.fori_loop(0, NCHUNK_ // 2, chunk_body, jnp.zeros((16,), jnp.int32))
    _wait(x_hbm, buf_v.at[0], sem0)
    _wait(x_hbm, buf_v.at[1], sem1)
    max_v[...] = maxv
    pltpu.sync_copy(hist_v, hist_hbm.at[wid])
    pltpu.sync_copy(max_v, max_hbm.at[wid])


def _cum_lanes(a):
    c = a
    sh = 1
    while sh < a.shape[-1]:
        z = jnp.zeros(a.shape[:-1] + (sh,), a.dtype)
        c = c + jnp.concatenate([z, c[..., :-sh]], axis=-1)
        sh *= 2
    return c


def _cum_rows(a):
    c = a
    sh = 1
    while sh < a.shape[0]:
        z = jnp.zeros((sh,) + a.shape[1:], a.dtype)
        c = c + jnp.concatenate([z, c[:-sh]], axis=0)
        sh *= 2
    return c


def _glue_body(hist_ref, max_ref, bufs_ref, o_ref):
    h = jnp.sum(hist_ref[...], axis=0)
    a = h.reshape(512, 128)
    rowcum = _cum_lanes(a)
    rs = rowcum[:, 127:]
    cum = rowcum + _cum_rows(rs) - rs
    maxu = jnp.max(max_ref[...])

    bits = [maxu]
    for kk in (K99_99_, K99_9_, K99_):
        b = jnp.sum((cum <= kk).astype(jnp.int32))
        bits.append((b << 15) | 0x4000)

    c = lax.broadcasted_iota(jnp.int32, (1, 128), 1)
    vbits = jnp.zeros((1, 128), jnp.int32)
    for t, w in enumerate(bits):
        vbits = jnp.where(c == t, w, vbits)
    vals = lax.bitcast_convert_type(vbits, jnp.float32)
    o_ref[...] = bufs_ref[...] * jnp.float32(G_) + vals * jnp.float32(1.0 - G_)


def _copy_body(x_ref, o_ref):
    o_ref[...] = x_ref[...]


def kernel(x, max_buf, p99_99_buf, p99_9_buf, p99_buf):
    xf = x.reshape(-1)
    hist, maxes = _sc_hist(xf)
    xr = x.reshape(8192, 2048)
    x_out = pl.pallas_call(
        _copy_body,
        grid=(16,),
        in_specs=[pl.BlockSpec((512, 2048), lambda i: (i, 0))],
        out_specs=pl.BlockSpec((512, 2048), lambda i: (i, 0)),
        out_shape=jax.ShapeDtypeStruct((8192, 2048), jnp.float32),
    )(xr).reshape(x.shape)
    bufs = jnp.zeros((1, 128), jnp.float32)
    bufs = bufs.at[0, 0].set(max_buf)
    bufs = bufs.at[0, 1].set(p99_99_buf)
    bufs = bufs.at[0, 2].set(p99_9_buf)
    bufs = bufs.at[0, 3].set(p99_buf)
    outv = pl.pallas_call(
        _glue_body,
        out_shape=jax.ShapeDtypeStruct((1, 128), jnp.float32),
    )(hist, maxes, bufs)
    return (x_out, outv[0, 0], outv[0, 1], outv[0, 2], outv[0, 3])


# speedup vs baseline: 3.8626x; 1.5298x over previous
import functools

import jax
import jax.numpy as jnp
from jax import lax
from jax.experimental import pallas as pl
from jax.experimental.pallas import tpu as pltpu
from jax.experimental.pallas import tpu_sc as plsc

G_ = 0.99
N_ = 2 * 4096 * 2048
NW_ = 32
ROWS_, COLS_ = 8192, 2048
RSHARD_ = ROWS_ // NW_
CROWS_ = 8
NCHUNK_ = RSHARD_ // CROWS_
HB_ = 65536

K99_99_ = int(round(0.9999 * N_)) - 1
K99_9_ = int(round(0.999 * N_)) - 1
K99_ = int(round(0.99 * N_)) - 1

_MESH = plsc.VectorSubcoreMesh(
    core_axis_name="c", subcore_axis_name="s", num_cores=2, num_subcores=16
)


def _zero_vmem(ref, nwords):
    z = jnp.zeros((16,), jnp.int32)

    def body(j, _):
        ref[pl.ds(j * 16, 16)] = z
        return 0

    lax.fori_loop(0, nwords // 16, body, 0, unroll=8)


def _start(x_hbm, buf_v, sem, base_row, ci):
    off = jnp.minimum(base_row + ci * CROWS_, ROWS_ - CROWS_)
    pltpu.async_copy(x_hbm.at[pl.ds(off, CROWS_), :], buf_v, sem)


def _wait(x_hbm, buf_v, sem):
    pltpu.make_async_copy(x_hbm.at[pl.ds(0, CROWS_), :], buf_v, sem).wait()


@functools.partial(
    pl.kernel,
    out_type=(
        jax.ShapeDtypeStruct((NW_, HB_), jnp.int32),
        jax.ShapeDtypeStruct((NW_, 16), jnp.int32),
    ),
    mesh=_MESH,
    compiler_params=pltpu.CompilerParams(needs_layout_passes=False),
    scratch_types=[
        pltpu.VMEM((2, CROWS_, COLS_), jnp.float32),
        pltpu.VMEM((HB_,), jnp.int32),
        pltpu.VMEM((16,), jnp.int32),
        pltpu.SemaphoreType.DMA,
        pltpu.SemaphoreType.DMA,
    ],
)
def _sc_hist(x_hbm, hist_hbm, max_hbm, buf_v, hist_v, max_v, sem0, sem1):
    wid = lax.axis_index("s") * 2 + lax.axis_index("c")
    base_row = wid * RSHARD_
    sems = (sem0, sem1)
    _start(x_hbm, buf_v.at[0], sem0, base_row, 0)
    _start(x_hbm, buf_v.at[1], sem1, base_row, 1)
    _zero_vmem(hist_v, HB_)
    ones = jnp.ones((16,), jnp.int32)

    def chunk_body(g, maxv):
        for b in range(2):
            ci = 2 * g + b
            _wait(x_hbm, buf_v.at[b], sems[b])
            for r in range(CROWS_):

                def vbody(i, mv, b=b, r=r):
                    v = buf_v[b, r, pl.ds(i * 16, 16)]
                    u = lax.bitcast_convert_type(v, jnp.int32) & 0x7FFFFFFF
                    plsc.addupdate_scatter(hist_v, [u >> 15], ones)
                    return jnp.maximum(mv, u)

                maxv = plsc.parallel_loop(
                    0, COLS_ // 16, 1, unroll=8, carry=maxv
                )(vbody)
            _start(x_hbm, buf_v.at[b], sems[b], base_row, ci + 2)
        return maxv

    maxv = lax.fori_loop(0, NCHUNK_ // 2, chunk_body, jnp.zeros((16,), jnp.int32))
    _wait(x_hbm, buf_v.at[0], sem0)
    _wait(x_hbm, buf_v.at[1], sem1)
    max_v[...] = maxv
    pltpu.sync_copy(hist_v, hist_hbm.at[wid])
    pltpu.sync_copy(max_v, max_hbm.at[wid])


def _cum_lanes(a):
    c = a
    sh = 1
    while sh < a.shape[-1]:
        z = jnp.zeros(a.shape[:-1] + (sh,), a.dtype)
        c = c + jnp.concatenate([z, c[..., :-sh]], axis=-1)
        sh *= 2
    return c


def _cum_rows(a):
    c = a
    sh = 1
    while sh < a.shape[0]:
        z = jnp.zeros((sh,) + a.shape[1:], a.dtype)
        c = c + jnp.concatenate([z, c[:-sh]], axis=0)
        sh *= 2
    return c


def _glue_body(hist_ref, max_ref, bufs_ref, o_ref):
    h = jnp.sum(hist_ref[...], axis=0)
    a = h.reshape(512, 128)
    rowcum = _cum_lanes(a)
    rs = rowcum[:, 127:]
    cum = rowcum + _cum_rows(rs) - rs
    maxu = jnp.max(max_ref[...])

    bits = [maxu]
    for kk in (K99_99_, K99_9_, K99_):
        b = jnp.sum((cum <= kk).astype(jnp.int32))
        bits.append((b << 15) | 0x4000)

    c = lax.broadcasted_iota(jnp.int32, (1, 128), 1)
    vbits = jnp.zeros((1, 128), jnp.int32)
    for t, w in enumerate(bits):
        vbits = jnp.where(c == t, w, vbits)
    vals = lax.bitcast_convert_type(vbits, jnp.float32)
    o_ref[...] = bufs_ref[...] * jnp.float32(G_) + vals * jnp.float32(1.0 - G_)


def _copy_body(x_ref, o_ref):
    o_ref[...] = x_ref[...]


def kernel(x, max_buf, p99_99_buf, p99_9_buf, p99_buf):
    xr = x.reshape(8192, 2048)
    hist, maxes = _sc_hist(xr)
    x_out = pl.pallas_call(
        _copy_body,
        grid=(16,),
        in_specs=[pl.BlockSpec((512, 2048), lambda i: (i, 0))],
        out_specs=pl.BlockSpec((512, 2048), lambda i: (i, 0)),
        out_shape=jax.ShapeDtypeStruct((8192, 2048), jnp.float32),
    )(xr).reshape(x.shape)
    bufs = jnp.zeros((1, 128), jnp.float32)
    bufs = bufs.at[0, 0].set(max_buf)
    bufs = bufs.at[0, 1].set(p99_99_buf)
    bufs = bufs.at[0, 2].set(p99_9_buf)
    bufs = bufs.at[0, 3].set(p99_buf)
    outv = pl.pallas_call(
        _glue_body,
        out_shape=jax.ShapeDtypeStruct((1, 128), jnp.float32),
    )(hist, maxes, bufs)
    return (x_out, outv[0, 0], outv[0, 1], outv[0, 2], outv[0, 3])


# speedup vs baseline: 3.8703x; 1.0020x over previous
import functools

import jax
import jax.numpy as jnp
from jax import lax
from jax.experimental import pallas as pl
from jax.experimental.pallas import tpu as pltpu
from jax.experimental.pallas import tpu_sc as plsc

G_ = 0.99
N_ = 2 * 4096 * 2048
NW_ = 32
ROWS_, COLS_ = 8192, 2048
RSHARD_ = ROWS_ // NW_
CROWS_ = 8
NCHUNK_ = RSHARD_ // CROWS_
HB_ = 65536

K99_99_ = int(round(0.9999 * N_)) - 1
K99_9_ = int(round(0.999 * N_)) - 1
K99_ = int(round(0.99 * N_)) - 1

_MESH = plsc.VectorSubcoreMesh(
    core_axis_name="c", subcore_axis_name="s", num_cores=2, num_subcores=16
)


def _zero_vmem(ref, nwords):
    z = jnp.zeros((16,), jnp.int32)

    def body(j, _):
        ref[pl.ds(j * 16, 16)] = z
        return 0

    lax.fori_loop(0, nwords // 16, body, 0, unroll=8)


def _start(x_hbm, buf_v, sem, base_row, ci):
    off = jnp.minimum(base_row + ci * CROWS_, ROWS_ - CROWS_)
    pltpu.async_copy(x_hbm.at[pl.ds(off, CROWS_), :], buf_v, sem)


def _wait(x_hbm, buf_v, sem):
    pltpu.make_async_copy(x_hbm.at[pl.ds(0, CROWS_), :], buf_v, sem).wait()


@functools.partial(
    pl.kernel,
    out_type=(
        jax.ShapeDtypeStruct((NW_, HB_), jnp.int32),
        jax.ShapeDtypeStruct((NW_, 16), jnp.int32),
    ),
    mesh=_MESH,
    compiler_params=pltpu.CompilerParams(needs_layout_passes=False),
    scratch_types=[
        pltpu.VMEM((2, CROWS_, COLS_), jnp.float32),
        pltpu.VMEM((HB_,), jnp.int32),
        pltpu.VMEM((16,), jnp.int32),
        pltpu.SemaphoreType.DMA,
        pltpu.SemaphoreType.DMA,
    ],
)
def _sc_hist(x_hbm, hist_hbm, max_hbm, buf_v, hist_v, max_v, sem0, sem1):
    wid = lax.axis_index("s") * 2 + lax.axis_index("c")
    base_row = wid * RSHARD_
    sems = (sem0, sem1)
    _start(x_hbm, buf_v.at[0], sem0, base_row, 0)
    _start(x_hbm, buf_v.at[1], sem1, base_row, 1)
    _zero_vmem(hist_v, HB_)
    ones = jnp.ones((16,), jnp.int32)

    def chunk_body(g, maxv):
        for b in range(2):
            ci = 2 * g + b
            _wait(x_hbm, buf_v.at[b], sems[b])
            for r in range(CROWS_):

                def vbody(i, mv, b=b, r=r):
                    v = buf_v[b, r, pl.ds(i * 16, 16)]
                    u = lax.bitcast_convert_type(v, jnp.int32) & 0x7FFFFFFF
                    plsc.addupdate_scatter(hist_v, [u >> 15], ones)
                    return jnp.maximum(mv, u)

                maxv = plsc.parallel_loop(
                    0, COLS_ // 16, 1, unroll=16, carry=maxv
                )(vbody)
            _start(x_hbm, buf_v.at[b], sems[b], base_row, ci + 2)
        return maxv

    maxv = lax.fori_loop(0, NCHUNK_ // 2, chunk_body, jnp.zeros((16,), jnp.int32))
    _wait(x_hbm, buf_v.at[0], sem0)
    _wait(x_hbm, buf_v.at[1], sem1)
    max_v[...] = maxv
    pltpu.sync_copy(max_v, max_hbm.at[wid])
    pltpu.sync_copy(hist_v, hist_hbm.at[wid])


def _cum_lanes(a):
    c = a
    sh = 1
    while sh < a.shape[-1]:
        z = jnp.zeros(a.shape[:-1] + (sh,), a.dtype)
        c = c + jnp.concatenate([z, c[..., :-sh]], axis=-1)
        sh *= 2
    return c


def _cum_rows(a):
    c = a
    sh = 1
    while sh < a.shape[0]:
        z = jnp.zeros((sh,) + a.shape[1:], a.dtype)
        c = c + jnp.concatenate([z, c[:-sh]], axis=0)
        sh *= 2
    return c


def _glue_body(hist_ref, max_ref, bufs_ref, o_ref):
    h = jnp.sum(hist_ref[...], axis=0)
    a = h.reshape(512, 128)
    rowcum = _cum_lanes(a)
    rs = rowcum[:, 127:]
    cum = rowcum + _cum_rows(rs) - rs
    maxu = jnp.max(max_ref[...])

    bits = [maxu]
    for kk in (K99_99_, K99_9_, K99_):
        b = jnp.sum((cum <= kk).astype(jnp.int32))
        bits.append((b << 15) | 0x4000)

    c = lax.broadcasted_iota(jnp.int32, (1, 128), 1)
    vbits = jnp.zeros((1, 128), jnp.int32)
    for t, w in enumerate(bits):
        vbits = jnp.where(c == t, w, vbits)
    vals = lax.bitcast_convert_type(vbits, jnp.float32)
    o_ref[...] = bufs_ref[...] * jnp.float32(G_) + vals * jnp.float32(1.0 - G_)


def _copy_body(x_ref, o_ref):
    o_ref[...] = x_ref[...]


def kernel(x, max_buf, p99_99_buf, p99_9_buf, p99_buf):
    xr = x.reshape(8192, 2048)
    hist, maxes = _sc_hist(xr)
    x_out = pl.pallas_call(
        _copy_body,
        grid=(16,),
        in_specs=[pl.BlockSpec((512, 2048), lambda i: (i, 0))],
        out_specs=pl.BlockSpec((512, 2048), lambda i: (i, 0)),
        out_shape=jax.ShapeDtypeStruct((8192, 2048), jnp.float32),
    )(xr).reshape(x.shape)
    bufs = jnp.zeros((1, 128), jnp.float32)
    bufs = bufs.at[0, 0].set(max_buf)
    bufs = bufs.at[0, 1].set(p99_99_buf)
    bufs = bufs.at[0, 2].set(p99_9_buf)
    bufs = bufs.at[0, 3].set(p99_buf)
    outv = pl.pallas_call(
        _glue_body,
        out_shape=jax.ShapeDtypeStruct((1, 128), jnp.float32),
    )(hist, maxes, bufs)
    return (x_out, outv[0, 0], outv[0, 1], outv[0, 2], outv[0, 3])
